# baseline (speedup 1.0000x reference)
import functools

import jax
import jax.numpy as jnp
import numpy as np
from jax import lax
from jax.experimental import pallas as pl
from jax.experimental.pallas import tpu as pltpu
from jax.experimental.pallas import tpu_sc as plsc

H = 16
B = 64
S = 4096
K = 4

_SAMPLED_HEAD = np.array(
    [7, 10, 9, 8, 6, 8, 1, 11, 10, 12, 5, 13, 13, 12, 7, 12], dtype=np.int32
)

def _argmax_flag_body(x_ref, s_ref, src_ref, cand_acc):
    h = pl.program_id(0)
    x = x_ref[0]
    nacc = 4
    per = (S // 128) // nacc
    accs = []
    for a in range(nacc):
        runv = x[:, 128 * per * a : 128 * (per * a + 1)]
        runi = jnp.full((B, 128), per * a, jnp.int32)
        for j in range(per * a + 1, per * (a + 1)):
            v = x[:, 128 * j : 128 * (j + 1)]
            gt = v > runv
            runv = jnp.where(gt, v, runv)
            runi = jnp.where(gt, j, runi)
        accs.append((runv, runi))
    while len(accs) > 1:
        (v1, i1), (v2, i2) = accs[0], accs[1]
        take1 = (v1 > v2) | ((v1 == v2) & (i1 < i2))
        accs = accs[2:] + [(jnp.where(take1, v1, v2), jnp.where(take1, i1, i2))]
    runv, runi = accs[0]
    m = jnp.max(runv, axis=-1, keepdims=True)
    col0 = lax.broadcasted_iota(jnp.int32, (B, 128), 1)
    fullidx = runi * 128 + col0
    first = jnp.min(jnp.where(runv == m, fullidx, S), axis=-1).astype(jnp.int32)
    cand_acc[pl.ds(h, 1), :] = first.reshape(1, B)

    @pl.when(h == H - 1)
    def _():
        c = cand_acc[...]
        counts = jnp.zeros((H, B), jnp.int32)
        for hp in range(H):
            counts = counts + (c == c[hp : hp + 1]).astype(jnp.int32)
        flag = jnp.max(counts) <= K
        iota = lax.broadcasted_iota(jnp.int32, (1, H), 1)
        src_ref[...] = jnp.where(flag, s_ref[...], iota)


def _tc_argmax_flag(x, s_const):
    return pl.pallas_call(
        _argmax_flag_body,
        grid=(H,),
        in_specs=[
            pl.BlockSpec((1, B, S), lambda h: (h, 0, 0)),
            pl.BlockSpec((1, H), lambda h: (0, 0)),
        ],
        out_specs=pl.BlockSpec((1, H), lambda h: (0, 0)),
        out_shape=jax.ShapeDtypeStruct((1, H), jnp.int32),
        scratch_shapes=[pltpu.VMEM((H, B), jnp.int32)],
    )(x, s_const)


_ROWS = 4
_NBUF = 6
_NCHUNK = (B // 2) // _ROWS


def _sc_copy_body(x_hbm, src_hbm, out_hbm, src_v, *bufsems):
    bufs = bufsems[:_NBUF]
    gsems = bufsems[_NBUF : 2 * _NBUF]
    ssems = bufsems[2 * _NBUF :]
    c = lax.axis_index("c")
    s = lax.axis_index("s")
    pltpu.sync_copy(src_hbm, src_v)
    src_s = src_v[0, pl.ds(s, 1)][0]
    b0 = c * (B // 2)

    def gather(g):
        return pltpu.async_copy(
            x_hbm.at[pl.ds(src_s, 1), pl.ds(b0 + _ROWS * g, _ROWS)],
            bufs[g % _NBUF],
            gsems[g % _NBUF],
        )

    def scatter(g):
        return pltpu.async_copy(
            bufs[g % _NBUF],
            out_hbm.at[pl.ds(s, 1), pl.ds(b0 + _ROWS * g, _ROWS)],
            ssems[g % _NBUF],
        )

    gathers = [None] * _NCHUNK
    scatters = [None] * _NCHUNK
    look = 3
    for g in range(min(look, _NCHUNK)):
        gathers[g] = gather(g)
    for g in range(_NCHUNK):
        if g + look - _NBUF >= 0:
            scatters[g + look - _NBUF].wait()
        if g + look < _NCHUNK:
            gathers[g + look] = gather(g + look)
        gathers[g].wait()
        scatters[g] = scatter(g)
    for g in range(max(0, _NCHUNK - look), _NCHUNK):
        scatters[g].wait()


def _sc_copy(x, src16):
    mesh = plsc.VectorSubcoreMesh(core_axis_name="c", subcore_axis_name="s")
    f = functools.partial(
        pl.kernel,
        mesh=mesh,
        out_type=jax.ShapeDtypeStruct((H, B, S), jnp.float32),
        scratch_types=[pltpu.VMEM((1, H), jnp.int32)]
        + [pltpu.VMEM((1, _ROWS, S), jnp.float32) for _ in range(_NBUF)]
        + [pltpu.SemaphoreType.DMA for _ in range(2 * _NBUF)],
    )(_sc_copy_body)
    return f(x, src16)


def kernel(attention_weight):
    x = attention_weight
    s_const = jnp.asarray(_SAMPLED_HEAD).reshape(1, H)
    src = _tc_argmax_flag(x, s_const)
    return _sc_copy(x, src)


# speedup vs baseline: 1.1256x; 1.1256x over previous
import functools

import jax
import jax.numpy as jnp
import numpy as np
from jax import lax
from jax.experimental import pallas as pl
from jax.experimental.pallas import tpu as pltpu
from jax.experimental.pallas import tpu_sc as plsc

H = 16
B = 64
S = 4096
K = 4

_SAMPLED_HEAD = np.array(
    [7, 10, 9, 8, 6, 8, 1, 11, 10, 12, 5, 13, 13, 12, 7, 12], dtype=np.int32
)

def _argmax_row_block(x):
    nacc = 4
    per = (S // 128) // nacc
    accs = []
    for a in range(nacc):
        runv = x[:, 128 * per * a : 128 * (per * a + 1)]
        runi = jnp.full((B, 128), per * a, jnp.int32)
        for j in range(per * a + 1, per * (a + 1)):
            v = x[:, 128 * j : 128 * (j + 1)]
            gt = v > runv
            runv = jnp.where(gt, v, runv)
            runi = jnp.where(gt, j, runi)
        accs.append((runv, runi))
    while len(accs) > 1:
        (v1, i1), (v2, i2) = accs[0], accs[1]
        take1 = (v1 > v2) | ((v1 == v2) & (i1 < i2))
        accs = accs[2:] + [(jnp.where(take1, v1, v2), jnp.where(take1, i1, i2))]
    runv, runi = accs[0]
    m = jnp.max(runv, axis=-1, keepdims=True)
    col0 = lax.broadcasted_iota(jnp.int32, (B, 128), 1)
    fullidx = runi * 128 + col0
    return jnp.min(jnp.where(runv == m, fullidx, S), axis=-1).astype(jnp.int32)


def _argmax_flag_body(xe_ref, xo_ref, s_ref, src_ref, cand_acc):
    h = pl.program_id(0)
    cand_acc[pl.ds(2 * h, 1), :] = _argmax_row_block(xe_ref[0]).reshape(1, B)
    cand_acc[pl.ds(2 * h + 1, 1), :] = _argmax_row_block(xo_ref[0]).reshape(1, B)

    @pl.when(h == H // 2 - 1)
    def _():
        c = cand_acc[...]
        counts = jnp.zeros((H, B), jnp.int32)
        for hp in range(H):
            counts = counts + (c == c[hp : hp + 1]).astype(jnp.int32)
        flag = jnp.max(counts) <= K
        iota = lax.broadcasted_iota(jnp.int32, (1, H), 1)
        src_ref[...] = jnp.where(flag, s_ref[...], iota)


def _tc_argmax_flag(x, s_const):
    return pl.pallas_call(
        _argmax_flag_body,
        grid=(H // 2,),
        in_specs=[
            pl.BlockSpec((1, B, S), lambda h: (2 * h, 0, 0)),
            pl.BlockSpec((1, B, S), lambda h: (2 * h + 1, 0, 0)),
            pl.BlockSpec((1, H), lambda h: (0, 0)),
        ],
        out_specs=pl.BlockSpec((1, H), lambda h: (0, 0)),
        out_shape=jax.ShapeDtypeStruct((1, H), jnp.int32),
        scratch_shapes=[pltpu.VMEM((H, B), jnp.int32)],
    )(x, x, s_const)


_ROWS = 4
_NBUF = 6
_NCHUNK = (B // 2) // _ROWS


def _sc_copy_body(x_hbm, src_hbm, out_hbm, src_v, *bufsems):
    bufs = bufsems[:_NBUF]
    gsems = bufsems[_NBUF : 2 * _NBUF]
    ssems = bufsems[2 * _NBUF :]
    c = lax.axis_index("c")
    s = lax.axis_index("s")
    pltpu.sync_copy(src_hbm, src_v)
    src_s = src_v[0, pl.ds(s, 1)][0]
    b0 = c * (B // 2)

    def gather(g):
        return pltpu.async_copy(
            x_hbm.at[pl.ds(src_s, 1), pl.ds(b0 + _ROWS * g, _ROWS)],
            bufs[g % _NBUF],
            gsems[g % _NBUF],
        )

    def scatter(g):
        return pltpu.async_copy(
            bufs[g % _NBUF],
            out_hbm.at[pl.ds(s, 1), pl.ds(b0 + _ROWS * g, _ROWS)],
            ssems[g % _NBUF],
        )

    gathers = [None] * _NCHUNK
    scatters = [None] * _NCHUNK
    look = 3
    for g in range(min(look, _NCHUNK)):
        gathers[g] = gather(g)
    for g in range(_NCHUNK):
        if g + look - _NBUF >= 0:
            scatters[g + look - _NBUF].wait()
        if g + look < _NCHUNK:
            gathers[g + look] = gather(g + look)
        gathers[g].wait()
        scatters[g] = scatter(g)
    for g in range(max(0, _NCHUNK - look), _NCHUNK):
        scatters[g].wait()


def _sc_copy(x, src16):
    mesh = plsc.VectorSubcoreMesh(core_axis_name="c", subcore_axis_name="s")
    f = functools.partial(
        pl.kernel,
        mesh=mesh,
        out_type=jax.ShapeDtypeStruct((H, B, S), jnp.float32),
        scratch_types=[pltpu.VMEM((1, H), jnp.int32)]
        + [pltpu.VMEM((1, _ROWS, S), jnp.float32) for _ in range(_NBUF)]
        + [pltpu.SemaphoreType.DMA for _ in range(2 * _NBUF)],
    )(_sc_copy_body)
    return f(x, src16)


def kernel(attention_weight):
    x = attention_weight
    s_const = jnp.asarray(_SAMPLED_HEAD).reshape(1, H)
    src = _tc_argmax_flag(x, s_const)
    return _sc_copy(x, src)


# speedup vs baseline: 1.1753x; 1.0442x over previous
import functools

import jax
import jax.numpy as jnp
import numpy as np
from jax import lax
from jax.experimental import pallas as pl
from jax.experimental.pallas import tpu as pltpu
from jax.experimental.pallas import tpu_sc as plsc

H = 16
B = 64
S = 4096
K = 4

_SAMPLED_HEAD = np.array(
    [7, 10, 9, 8, 6, 8, 1, 11, 10, 12, 5, 13, 13, 12, 7, 12], dtype=np.int32
)

def _argmax_row_block(x):
    nacc = 4
    per = (S // 128) // nacc
    accs = []
    for a in range(nacc):
        runv = x[:, 128 * per * a : 128 * (per * a + 1)]
        runi = jnp.full((B, 128), per * a, jnp.int32)
        for j in range(per * a + 1, per * (a + 1)):
            v = x[:, 128 * j : 128 * (j + 1)]
            gt = v > runv
            runv = jnp.where(gt, v, runv)
            runi = jnp.where(gt, j, runi)
        accs.append((runv, runi))
    while len(accs) > 1:
        (v1, i1), (v2, i2) = accs[0], accs[1]
        take1 = (v1 > v2) | ((v1 == v2) & (i1 < i2))
        accs = accs[2:] + [(jnp.where(take1, v1, v2), jnp.where(take1, i1, i2))]
    runv, runi = accs[0]
    m = jnp.max(runv, axis=-1, keepdims=True)
    col0 = lax.broadcasted_iota(jnp.int32, (B, 128), 1)
    fullidx = runi * 128 + col0
    return jnp.min(jnp.where(runv == m, fullidx, S), axis=-1).astype(jnp.int32)


def _argmax_flag_body(x0_ref, x1_ref, x2_ref, x3_ref, s_ref, src_ref, cand_acc):
    h = pl.program_id(0)
    for q, xr in enumerate((x0_ref, x1_ref, x2_ref, x3_ref)):
        cand_acc[pl.ds(4 * h + q, 1), :] = _argmax_row_block(xr[0]).reshape(1, B)

    @pl.when(h == H // 4 - 1)
    def _():
        c = cand_acc[...]
        counts = jnp.zeros((H, B), jnp.int32)
        for hp in range(H):
            counts = counts + (c == c[hp : hp + 1]).astype(jnp.int32)
        flag = jnp.max(counts) <= K
        iota = lax.broadcasted_iota(jnp.int32, (1, H), 1)
        src_ref[...] = jnp.where(flag, s_ref[...], iota)


def _tc_argmax_flag(x, s_const):
    return pl.pallas_call(
        _argmax_flag_body,
        grid=(H // 4,),
        in_specs=[
            pl.BlockSpec((1, B, S), lambda h: (4 * h, 0, 0)),
            pl.BlockSpec((1, B, S), lambda h: (4 * h + 1, 0, 0)),
            pl.BlockSpec((1, B, S), lambda h: (4 * h + 2, 0, 0)),
            pl.BlockSpec((1, B, S), lambda h: (4 * h + 3, 0, 0)),
            pl.BlockSpec((1, H), lambda h: (0, 0)),
        ],
        out_specs=pl.BlockSpec((1, H), lambda h: (0, 0)),
        out_shape=jax.ShapeDtypeStruct((1, H), jnp.int32),
        scratch_shapes=[pltpu.VMEM((H, B), jnp.int32)],
    )(x, x, x, x, s_const)


_ROWS = 4
_NBUF = 6
_NCHUNK = (B // 2) // _ROWS


def _sc_copy_body(x_hbm, src_hbm, out_hbm, src_v, *bufsems):
    bufs = bufsems[:_NBUF]
    gsems = bufsems[_NBUF : 2 * _NBUF]
    ssems = bufsems[2 * _NBUF :]
    c = lax.axis_index("c")
    s = lax.axis_index("s")
    pltpu.sync_copy(src_hbm, src_v)
    src_s = src_v[0, pl.ds(s, 1)][0]
    b0 = c * (B // 2)

    def gather(g):
        return pltpu.async_copy(
            x_hbm.at[pl.ds(src_s, 1), pl.ds(b0 + _ROWS * g, _ROWS)],
            bufs[g % _NBUF],
            gsems[g % _NBUF],
        )

    def scatter(g):
        return pltpu.async_copy(
            bufs[g % _NBUF],
            out_hbm.at[pl.ds(s, 1), pl.ds(b0 + _ROWS * g, _ROWS)],
            ssems[g % _NBUF],
        )

    gathers = [None] * _NCHUNK
    scatters = [None] * _NCHUNK
    look = 3
    for g in range(min(look, _NCHUNK)):
        gathers[g] = gather(g)
    for g in range(_NCHUNK):
        if g + look - _NBUF >= 0:
            scatters[g + look - _NBUF].wait()
        if g + look < _NCHUNK:
            gathers[g + look] = gather(g + look)
        gathers[g].wait()
        scatters[g] = scatter(g)
    for g in range(max(0, _NCHUNK - look), _NCHUNK):
        scatters[g].wait()


def _sc_copy(x, src16):
    mesh = plsc.VectorSubcoreMesh(core_axis_name="c", subcore_axis_name="s")
    f = functools.partial(
        pl.kernel,
        mesh=mesh,
        out_type=jax.ShapeDtypeStruct((H, B, S), jnp.float32),
        scratch_types=[pltpu.VMEM((1, H), jnp.int32)]
        + [pltpu.VMEM((1, _ROWS, S), jnp.float32) for _ in range(_NBUF)]
        + [pltpu.SemaphoreType.DMA for _ in range(2 * _NBUF)],
    )(_sc_copy_body)
    return f(x, src16)


def kernel(attention_weight):
    x = attention_weight
    s_const = jnp.asarray(_SAMPLED_HEAD).reshape(1, H)
    src = _tc_argmax_flag(x, s_const)
    return _sc_copy(x, src)


# speedup vs baseline: 1.1919x; 1.0141x over previous
import functools

import jax
import jax.numpy as jnp
import numpy as np
from jax import lax
from jax.experimental import pallas as pl
from jax.experimental.pallas import tpu as pltpu
from jax.experimental.pallas import tpu_sc as plsc

H = 16
B = 64
S = 4096
K = 4

_SAMPLED_HEAD = np.array(
    [7, 10, 9, 8, 6, 8, 1, 11, 10, 12, 5, 13, 13, 12, 7, 12], dtype=np.int32
)

def _argmax_row_block(x):
    nacc = 4
    per = (S // 128) // nacc
    accs = []
    for a in range(nacc):
        runv = x[:, 128 * per * a : 128 * (per * a + 1)]
        runi = jnp.full((B, 128), per * a, jnp.int32)
        for j in range(per * a + 1, per * (a + 1)):
            v = x[:, 128 * j : 128 * (j + 1)]
            gt = v > runv
            runv = jnp.where(gt, v, runv)
            runi = jnp.where(gt, j, runi)
        accs.append((runv, runi))
    while len(accs) > 1:
        (v1, i1), (v2, i2) = accs[0], accs[1]
        take1 = (v1 > v2) | ((v1 == v2) & (i1 < i2))
        accs = accs[2:] + [(jnp.where(take1, v1, v2), jnp.where(take1, i1, i2))]
    runv, runi = accs[0]
    m = jnp.max(runv, axis=-1, keepdims=True)
    col0 = lax.broadcasted_iota(jnp.int32, (B, 128), 1)
    fullidx = runi * 128 + col0
    return jnp.min(jnp.where(runv == m, fullidx, S), axis=-1).astype(jnp.int32)


_NSTREAM = 8


def _argmax_flag_body(*refs):
    xrefs = refs[:_NSTREAM]
    s_ref, src_ref, cand_acc = refs[_NSTREAM:]
    h = pl.program_id(0)
    for q, xr in enumerate(xrefs):
        cand_acc[pl.ds(_NSTREAM * h + q, 1), :] = _argmax_row_block(xr[0]).reshape(1, B)

    @pl.when(h == H // _NSTREAM - 1)
    def _():
        c = cand_acc[...]
        counts = jnp.zeros((H, B), jnp.int32)
        for hp in range(H):
            counts = counts + (c == c[hp : hp + 1]).astype(jnp.int32)
        flag = jnp.max(counts) <= K
        iota = lax.broadcasted_iota(jnp.int32, (1, H), 1)
        src_ref[...] = jnp.where(flag, s_ref[...], iota)


def _tc_argmax_flag(x, s_const):
    def _mk_spec(q):
        return pl.BlockSpec((1, B, S), lambda h: (_NSTREAM * h + q, 0, 0))

    return pl.pallas_call(
        _argmax_flag_body,
        grid=(H // _NSTREAM,),
        in_specs=[_mk_spec(q) for q in range(_NSTREAM)]
        + [pl.BlockSpec((1, H), lambda h: (0, 0))],
        out_specs=pl.BlockSpec((1, H), lambda h: (0, 0)),
        out_shape=jax.ShapeDtypeStruct((1, H), jnp.int32),
        scratch_shapes=[pltpu.VMEM((H, B), jnp.int32)],
    )(*([x] * _NSTREAM), s_const)


_ROWS = 4
_NBUF = 6
_NCHUNK = (B // 2) // _ROWS


def _sc_copy_body(x_hbm, src_hbm, out_hbm, src_v, *bufsems):
    bufs = bufsems[:_NBUF]
    gsems = bufsems[_NBUF : 2 * _NBUF]
    ssems = bufsems[2 * _NBUF :]
    c = lax.axis_index("c")
    s = lax.axis_index("s")
    pltpu.sync_copy(src_hbm, src_v)
    src_s = src_v[0, pl.ds(s, 1)][0]
    b0 = c * (B // 2)

    def gather(g):
        return pltpu.async_copy(
            x_hbm.at[pl.ds(src_s, 1), pl.ds(b0 + _ROWS * g, _ROWS)],
            bufs[g % _NBUF],
            gsems[g % _NBUF],
        )

    def scatter(g):
        return pltpu.async_copy(
            bufs[g % _NBUF],
            out_hbm.at[pl.ds(s, 1), pl.ds(b0 + _ROWS * g, _ROWS)],
            ssems[g % _NBUF],
        )

    gathers = [None] * _NCHUNK
    scatters = [None] * _NCHUNK
    look = 3
    for g in range(min(look, _NCHUNK)):
        gathers[g] = gather(g)
    for g in range(_NCHUNK):
        if g + look - _NBUF >= 0:
            scatters[g + look - _NBUF].wait()
        if g + look < _NCHUNK:
            gathers[g + look] = gather(g + look)
        gathers[g].wait()
        scatters[g] = scatter(g)
    for g in range(max(0, _NCHUNK - look), _NCHUNK):
        scatters[g].wait()


def _sc_copy(x, src16):
    mesh = plsc.VectorSubcoreMesh(core_axis_name="c", subcore_axis_name="s")
    f = functools.partial(
        pl.kernel,
        mesh=mesh,
        out_type=jax.ShapeDtypeStruct((H, B, S), jnp.float32),
        scratch_types=[pltpu.VMEM((1, H), jnp.int32)]
        + [pltpu.VMEM((1, _ROWS, S), jnp.float32) for _ in range(_NBUF)]
        + [pltpu.SemaphoreType.DMA for _ in range(2 * _NBUF)],
    )(_sc_copy_body)
    return f(x, src16)


def kernel(attention_weight):
    x = attention_weight
    s_const = jnp.asarray(_SAMPLED_HEAD).reshape(1, H)
    src = _tc_argmax_flag(x, s_const)
    return _sc_copy(x, src)
